# Initial kernel scaffold; baseline (speedup 1.0000x reference)
import jax
import jax.numpy as jnp
from jax.experimental import pallas as pl


def kernel(wav, offsets):
    raise NotImplementedError("write your pallas kernel here")


# speedup vs baseline: 2.7787x; 2.7787x over previous
import jax
import jax.numpy as jnp
from jax import lax
from jax.experimental import pallas as pl
from jax.experimental.pallas import tpu as pltpu
from jax.experimental.pallas import tpu_sc as plsc

SHIFT = 8192
TIME = 441000
LENGTH = TIME - SHIFT
ROWS = 16 * 4 * 2

NUM_CORES = 2
NUM_SUBCORES = 16
NUM_WORKERS = NUM_CORES * NUM_SUBCORES
ROWS_PER_WORKER = ROWS // NUM_WORKERS

CHUNK = 32768
N_FULL = LENGTH // CHUNK
REM = LENGTH - N_FULL * CHUNK
LANES = 16


def _shift_body(wav_hbm, off_hbm, out_hbm, off_v, in_buf, out_buf):
    wid = lax.axis_index("s") * NUM_CORES + lax.axis_index("c")
    pltpu.sync_copy(off_hbm, off_v)
    lane_iota = lax.iota(jnp.int32, LANES)

    for j in range(ROWS_PER_WORKER):
        row = wid * ROWS_PER_WORKER + j
        offv = plsc.load_gather(off_v, [jnp.full((LANES,), row, jnp.int32)])
        off_s = jnp.max(offv)
        r = lax.bitwise_and(off_s, 7)
        a = pl.multiple_of(off_s - r, 8)
        src_base = pl.multiple_of(row * TIME + a, 8)
        dst_base = row * LENGTH
        base_idx = lane_iota + r

        def do_chunk(src_off, dst_off, out_len, nvec):
            in_len = out_len + 8
            pltpu.sync_copy(
                wav_hbm.at[pl.ds(src_off, in_len)],
                in_buf.at[pl.ds(0, in_len)],
            )

            @plsc.parallel_loop(0, nvec, unroll=8)
            def _gather(i):
                idx = base_idx + i * LANES
                out_buf[pl.ds(i * LANES, LANES)] = plsc.load_gather(
                    in_buf, [idx]
                )

            pltpu.sync_copy(
                out_buf.at[pl.ds(0, out_len)],
                out_hbm.at[pl.ds(dst_off, out_len)],
            )

        for k in range(N_FULL):
            do_chunk(
                pl.multiple_of(src_base + k * CHUNK, 8),
                pl.multiple_of(dst_base + k * CHUNK, 8),
                CHUNK,
                CHUNK // LANES,
            )
        do_chunk(
            pl.multiple_of(src_base + N_FULL * CHUNK, 8),
            pl.multiple_of(dst_base + N_FULL * CHUNK, 8),
            REM,
            (REM + 8) // LANES,
        )


def kernel(wav, offsets):
    batch, sources, channels, time = wav.shape
    wav_flat = wav.reshape(ROWS * TIME)
    offs = jnp.broadcast_to(
        offsets.reshape(batch * sources, 1), (batch * sources, channels)
    ).reshape(ROWS)
    shift = pl.kernel(
        _shift_body,
        out_type=jax.ShapeDtypeStruct((ROWS * LENGTH,), jnp.float32),
        mesh=plsc.VectorSubcoreMesh(core_axis_name="c", subcore_axis_name="s"),
        compiler_params=pltpu.CompilerParams(needs_layout_passes=False),
        scratch_types=[
            pltpu.VMEM((ROWS,), jnp.int32),
            pltpu.VMEM((CHUNK + 8,), jnp.float32),
            pltpu.VMEM((CHUNK,), jnp.float32),
        ],
    )
    out = shift(wav_flat, offs)
    return out.reshape(batch, sources, channels, LENGTH)


# speedup vs baseline: 53.4268x; 19.2271x over previous
import jax
import jax.numpy as jnp
from jax import lax
from jax.experimental import pallas as pl
from jax.experimental.pallas import tpu as pltpu
from jax.experimental.pallas import tpu_sc as plsc

SHIFT = 8192
TIME = 441000
LENGTH = TIME - SHIFT
GROUPS = 16 * 4
CHANNELS = 2

NUM_CORES = 2
NUM_SUBCORES = 16
NUM_WORKERS = NUM_CORES * NUM_SUBCORES
GROUPS_PER_WORKER = GROUPS // NUM_WORKERS

CHUNK = 14720
N_FULL = LENGTH // CHUNK
REM = LENGTH - N_FULL * CHUNK
N_CHUNKS = N_FULL + 1
LANES = 16


def _shift_body(wav_hbm, off_hbm, out_hbm, off_v,
                in_buf0, in_buf1, out_buf0, out_buf1,
                sem_in0, sem_in1, sem_out0, sem_out1):
    in_bufs = (in_buf0, in_buf1)
    out_bufs = (out_buf0, out_buf1)
    sems_in = (sem_in0, sem_in1)
    sems_out = (sem_out0, sem_out1)

    wid = lax.axis_index("s") * NUM_CORES + lax.axis_index("c")
    pltpu.sync_copy(off_hbm, off_v)
    lane_iota = lax.iota(jnp.int32, LANES)

    groups = []
    for j in range(GROUPS_PER_WORKER):
        g = wid * GROUPS_PER_WORKER + j
        offv = plsc.load_gather(off_v, [jnp.full((LANES,), g, jnp.int32)])
        off_s = jnp.max(offv)
        r = lax.bitwise_and(off_s, 127)
        a = pl.multiple_of(off_s - r, 128)
        groups.append((g, a, lane_iota + r))

    items = []
    for g, a, base_idx in groups:
        for k in range(N_CHUNKS):
            out_len = CHUNK if k < N_FULL else -(-REM // 128) * 128
            nvec = CHUNK // LANES if k < N_FULL else (REM + LANES - 1) // LANES
            items.append((
                g,
                pl.multiple_of(a + k * CHUNK, 128),
                k * CHUNK,
                out_len,
                nvec,
                base_idx,
            ))
    n = len(items)

    def start_in(i):
        g, src_off, _, out_len, _, _ = items[i]
        in_len = -(-(out_len + 128) // 128) * 128
        cp = pltpu.make_async_copy(
            wav_hbm.at[g, :, pl.ds(src_off, in_len)],
            in_bufs[i % 2].at[:, pl.ds(0, in_len)],
            sems_in[i % 2],
        )
        cp.start()
        return cp

    out_cps = [None] * n
    in_cps = [None] * n
    in_cps[0] = start_in(0)
    if n > 1:
        in_cps[1] = start_in(1)

    for i in range(n):
        g, _, dst_off, out_len, nvec, base_idx = items[i]
        ib, ob = in_bufs[i % 2], out_bufs[i % 2]
        in_cps[i].wait()
        if i >= 2:
            out_cps[i - 2].wait()

        @plsc.parallel_loop(0, nvec, unroll=8)
        def _gather(v):
            idx = base_idx + v * LANES
            ob[0, pl.ds(v * LANES, LANES)] = plsc.load_gather(
                ib, [jnp.zeros((LANES,), jnp.int32), idx])
            ob[1, pl.ds(v * LANES, LANES)] = plsc.load_gather(
                ib, [jnp.ones((LANES,), jnp.int32), idx])

        cp = pltpu.make_async_copy(
            ob.at[:, pl.ds(0, out_len)],
            out_hbm.at[g, :, pl.ds(pl.multiple_of(dst_off, 128), out_len)],
            sems_out[i % 2],
        )
        cp.start()
        out_cps[i] = cp
        if i + 2 < n:
            in_cps[i + 2] = start_in(i + 2)

    out_cps[n - 2].wait()
    out_cps[n - 1].wait()


def kernel(wav, offsets):
    batch, sources, channels, time = wav.shape
    wav3 = wav.reshape(GROUPS, CHANNELS, TIME)
    offs = offsets.reshape(GROUPS)
    shift = pl.kernel(
        _shift_body,
        out_type=jax.ShapeDtypeStruct((GROUPS, CHANNELS, LENGTH), jnp.float32),
        mesh=plsc.VectorSubcoreMesh(core_axis_name="c", subcore_axis_name="s"),
        compiler_params=pltpu.CompilerParams(needs_layout_passes=False),
        scratch_types=[
            pltpu.VMEM((GROUPS,), jnp.int32),
            pltpu.VMEM((CHANNELS, CHUNK + 128), jnp.float32),
            pltpu.VMEM((CHANNELS, CHUNK + 128), jnp.float32),
            pltpu.VMEM((CHANNELS, CHUNK), jnp.float32),
            pltpu.VMEM((CHANNELS, CHUNK), jnp.float32),
            pltpu.SemaphoreType.DMA,
            pltpu.SemaphoreType.DMA,
            pltpu.SemaphoreType.DMA,
            pltpu.SemaphoreType.DMA,
        ],
    )
    out = shift(wav3, offs)
    return out.reshape(batch, sources, channels, LENGTH)
